# baseline (speedup 1.0000x reference)
import functools

import jax
import jax.numpy as jnp
from jax import lax
from jax.experimental import pallas as pl
from jax.experimental.pallas import tpu as pltpu
from jax.experimental.pallas import tpu_sc as plsc

_B, _N, _F, _M, _H = 4, 4096, 64, 64, 8
_MH = _M * _H
_TR = 1024
_NW = 32
_RPW = _N // _NW


def _sc_diag_body(aflat_ref, diag_ref, idxv, diagv, sem):
    wid = lax.axis_index("c") * 16 + lax.axis_index("s")
    base_row = wid * _RPW
    iota16 = lax.iota(jnp.int32, 16)
    for b in range(_B):
        for k in range(_RPW // 16):
            g = base_row + k * 16 + iota16
            off = pl.multiple_of(k * 16, 16)
            idxv[pl.ds(off, 16)] = b * (_N * _N) + g * (_N + 1)
        pltpu.async_copy(aflat_ref.at[idxv], diagv, sem).wait()
        pltpu.sync_copy(diagv, diag_ref.at[b, wid])


_sc_diag = functools.partial(
    pl.kernel,
    out_type=jax.ShapeDtypeStruct((_B, _NW, _RPW), jnp.float32),
    mesh=plsc.VectorSubcoreMesh(core_axis_name="c", subcore_axis_name="s"),
    scratch_types=[
        pltpu.VMEM((_RPW,), jnp.int32),
        pltpu.VMEM((_RPW,), jnp.float32),
        pltpu.SemaphoreType.DMA,
    ],
)(_sc_diag_body)


def _phase1_body(a_ref, stats_ref):
    r = pl.program_id(1)
    a = a_ref[0]
    rowsum = jnp.sum(a, axis=1)
    csum = jnp.sum(a, axis=0)

    @pl.when(r == 0)
    def _():
        stats_ref[0, :, 0:1] = jnp.zeros((_N, 1), jnp.float32)

    stats_ref[0, :, 0:1] += csum[:, None]
    stats_ref[0, pl.ds(r * _TR, _TR), 1:2] = rowsum[:, None]


def _phase2_body(x_ref, stats_ref, diag_ref, wn_ref, w2n_ref, wa_ref,
                 sel_ref, binv_ref, psi_ref):
    inv_n = 1.0 / _N
    x = x_ref[0]
    h1 = jnp.dot(x, wn_ref[...], preferred_element_type=jnp.float32)
    sumx = jnp.sum(x, axis=0, keepdims=True)
    base = jnp.dot(sumx, w2n_ref[...], preferred_element_type=jnp.float32)
    cl = stats_ref[0, :, 0:1]
    rw = stats_ref[0, :, 1:2]
    dg = diag_ref[0]
    sum_diag = jnp.sum(dg)
    suma = jnp.sum(rw)
    wa = wa_ref[...]
    base = (base + (sum_diag * inv_n) * wa[1:2]
            + (suma * inv_n * inv_n) * wa[4:5] + wa[5:6])
    pernode = (dg * wa[0:1] + (rw * inv_n) * wa[2:3]
               + (cl * inv_n) * wa[3:4])
    hidden = jnp.maximum(h1 + pernode + base, 0.0)
    s = jnp.sum(hidden, axis=0, keepdims=True)
    psi = jnp.dot(s, sel_ref[...], preferred_element_type=jnp.float32) * inv_n
    psi_ref[0, 0, :] = psi[0] + binv_ref[0]


def kernel(X, A, W_eq, b_eq, W_inv, b_inv):
    n = float(_N)
    Wx = W_eq[:, :, :_F, :]
    wav = W_eq[:, :, _F, :]
    Wn = (Wx[:, 0] + (Wx[:, 2] + Wx[:, 3]) * (1.0 / n))
    Wn = jnp.transpose(Wn, (1, 0, 2)).reshape(_F, _MH)
    W2n = (Wx[:, 1] * (1.0 / n) + Wx[:, 4] * (1.0 / (n * n)))
    W2n = jnp.transpose(W2n, (1, 0, 2)).reshape(_F, _MH)
    wa_rows = [wav[:, p].reshape(_MH) for p in range(5)]
    wa_pack = jnp.stack(wa_rows + [b_eq.reshape(_MH),
                                   jnp.zeros((_MH,), jnp.float32),
                                   jnp.zeros((_MH,), jnp.float32)])
    mh_ids = jnp.arange(_MH, dtype=jnp.int32) // _H
    sel = jnp.where(mh_ids[:, None] == jnp.arange(_M, dtype=jnp.int32)[None, :],
                    W_inv.reshape(_MH)[:, None], 0.0)

    diag = _sc_diag(A.reshape(_B * _N * _N))
    diag = diag.reshape(_B, _N, 1)

    stats = pl.pallas_call(
        _phase1_body,
        grid=(_B, _N // _TR),
        in_specs=[pl.BlockSpec((1, _TR, _N), lambda b, r: (b, r, 0))],
        out_specs=pl.BlockSpec((1, _N, 8), lambda b, r: (b, 0, 0)),
        out_shape=jax.ShapeDtypeStruct((_B, _N, 8), jnp.float32),
    )(A)

    psi = pl.pallas_call(
        _phase2_body,
        grid=(_B,),
        in_specs=[
            pl.BlockSpec((1, _N, _F), lambda b: (b, 0, 0)),
            pl.BlockSpec((1, _N, 8), lambda b: (b, 0, 0)),
            pl.BlockSpec((1, _N, 1), lambda b: (b, 0, 0)),
            pl.BlockSpec((_F, _MH), lambda b: (0, 0)),
            pl.BlockSpec((_F, _MH), lambda b: (0, 0)),
            pl.BlockSpec((8, _MH), lambda b: (0, 0)),
            pl.BlockSpec((_MH, _M), lambda b: (0, 0)),
            pl.BlockSpec((1, _M), lambda b: (0, 0)),
        ],
        out_specs=pl.BlockSpec((1, 1, _M), lambda b: (b, 0, 0)),
        out_shape=jax.ShapeDtypeStruct((_B, 1, _M), jnp.float32),
    )(X, stats, diag, Wn, W2n, wa_pack, sel, b_inv.reshape(1, _M))
    return psi.reshape(_B, _M)


# speedup vs baseline: 2.2756x; 2.2756x over previous
import functools

import jax
import jax.numpy as jnp
from jax import lax
from jax.experimental import pallas as pl
from jax.experimental.pallas import tpu as pltpu
from jax.experimental.pallas import tpu_sc as plsc

_B, _N, _F, _M, _H = 4, 4096, 64, 64, 8
_MH = _M * _H
_TR = 1024
_NW = 32
_RPW = _N // _NW


def _sc_diag_body(a_ref, diag_ref, dbuf, diagv, sem):
    wid = lax.axis_index("c") * 16 + lax.axis_index("s")
    r0 = pl.multiple_of(wid * _RPW, _RPW)
    iota16 = lax.iota(jnp.int32, 16)
    zero = jnp.zeros((16,), jnp.float32)

    def per_batch(b, carry):
        pltpu.async_copy(a_ref.at[b, pl.ds(r0, _RPW), pl.ds(r0, _RPW)],
                         dbuf, sem).wait()
        for k in range(_RPW // 16):
            dvec = zero
            for i in range(16):
                v = dbuf[k * 16 + i, pl.ds(k * 16, 16)]
                dvec = dvec + jnp.where(iota16 == i, v, 0.0)
            diagv[pl.ds(k * 16, 16)] = dvec
        pltpu.sync_copy(diagv, diag_ref.at[b, wid])
        return carry

    lax.fori_loop(0, _B, per_batch, 0)


_sc_diag = functools.partial(
    pl.kernel,
    out_type=jax.ShapeDtypeStruct((_B, _NW, _RPW), jnp.float32),
    mesh=plsc.VectorSubcoreMesh(core_axis_name="c", subcore_axis_name="s"),
    scratch_types=[
        pltpu.VMEM((_RPW, _RPW), jnp.float32),
        pltpu.VMEM((_RPW,), jnp.float32),
        pltpu.SemaphoreType.DMA,
    ],
)(_sc_diag_body)


def _phase1_body(a_ref, stats_ref):
    r = pl.program_id(1)
    a = a_ref[0]
    rowsum = jnp.sum(a, axis=1)
    csum = jnp.sum(a, axis=0)

    @pl.when(r == 0)
    def _():
        stats_ref[0, :, 0:1] = jnp.zeros((_N, 1), jnp.float32)

    stats_ref[0, :, 0:1] += csum[:, None]
    stats_ref[0, pl.ds(r * _TR, _TR), 1:2] = rowsum[:, None]


def _phase2_body(x_ref, stats_ref, diag_ref, wn_ref, w2n_ref, wa_ref,
                 sel_ref, binv_ref, psi_ref):
    inv_n = 1.0 / _N
    x = x_ref[0]
    h1 = jnp.dot(x, wn_ref[...], preferred_element_type=jnp.float32)
    sumx = jnp.sum(x, axis=0, keepdims=True)
    base = jnp.dot(sumx, w2n_ref[...], preferred_element_type=jnp.float32)
    cl = stats_ref[0, :, 0:1]
    rw = stats_ref[0, :, 1:2]
    dg = diag_ref[0]
    sum_diag = jnp.sum(dg)
    suma = jnp.sum(rw)
    wa = wa_ref[...]
    base = (base + (sum_diag * inv_n) * wa[1:2]
            + (suma * inv_n * inv_n) * wa[4:5] + wa[5:6])
    pernode = (dg * wa[0:1] + (rw * inv_n) * wa[2:3]
               + (cl * inv_n) * wa[3:4])
    hidden = jnp.maximum(h1 + pernode + base, 0.0)
    s = jnp.sum(hidden, axis=0, keepdims=True)
    psi = jnp.dot(s, sel_ref[...], preferred_element_type=jnp.float32) * inv_n
    psi_ref[0, 0, :] = psi[0] + binv_ref[0]


def kernel(X, A, W_eq, b_eq, W_inv, b_inv):
    n = float(_N)
    Wx = W_eq[:, :, :_F, :]
    wav = W_eq[:, :, _F, :]
    Wn = (Wx[:, 0] + (Wx[:, 2] + Wx[:, 3]) * (1.0 / n))
    Wn = jnp.transpose(Wn, (1, 0, 2)).reshape(_F, _MH)
    W2n = (Wx[:, 1] * (1.0 / n) + Wx[:, 4] * (1.0 / (n * n)))
    W2n = jnp.transpose(W2n, (1, 0, 2)).reshape(_F, _MH)
    wa_rows = [wav[:, p].reshape(_MH) for p in range(5)]
    wa_pack = jnp.stack(wa_rows + [b_eq.reshape(_MH),
                                   jnp.zeros((_MH,), jnp.float32),
                                   jnp.zeros((_MH,), jnp.float32)])
    mh_ids = jnp.arange(_MH, dtype=jnp.int32) // _H
    sel = jnp.where(mh_ids[:, None] == jnp.arange(_M, dtype=jnp.int32)[None, :],
                    W_inv.reshape(_MH)[:, None], 0.0)

    diag = _sc_diag(A)
    diag = diag.reshape(_B, _N, 1)

    stats = pl.pallas_call(
        _phase1_body,
        grid=(_B, _N // _TR),
        in_specs=[pl.BlockSpec((1, _TR, _N), lambda b, r: (b, r, 0))],
        out_specs=pl.BlockSpec((1, _N, 8), lambda b, r: (b, 0, 0)),
        out_shape=jax.ShapeDtypeStruct((_B, _N, 8), jnp.float32),
    )(A)

    psi = pl.pallas_call(
        _phase2_body,
        grid=(_B,),
        in_specs=[
            pl.BlockSpec((1, _N, _F), lambda b: (b, 0, 0)),
            pl.BlockSpec((1, _N, 8), lambda b: (b, 0, 0)),
            pl.BlockSpec((1, _N, 1), lambda b: (b, 0, 0)),
            pl.BlockSpec((_F, _MH), lambda b: (0, 0)),
            pl.BlockSpec((_F, _MH), lambda b: (0, 0)),
            pl.BlockSpec((8, _MH), lambda b: (0, 0)),
            pl.BlockSpec((_MH, _M), lambda b: (0, 0)),
            pl.BlockSpec((1, _M), lambda b: (0, 0)),
        ],
        out_specs=pl.BlockSpec((1, 1, _M), lambda b: (b, 0, 0)),
        out_shape=jax.ShapeDtypeStruct((_B, 1, _M), jnp.float32),
    )(X, stats, diag, Wn, W2n, wa_pack, sel, b_inv.reshape(1, _M))
    return psi.reshape(_B, _M)


# speedup vs baseline: 2.4392x; 1.0719x over previous
import functools

import jax
import jax.numpy as jnp
from jax import lax
from jax.experimental import pallas as pl
from jax.experimental.pallas import tpu as pltpu
from jax.experimental.pallas import tpu_sc as plsc

_B, _N, _F, _M, _H = 4, 4096, 64, 64, 8
_MH = _M * _H
_TR = 1024
_NW = 32
_RPW = _N // _NW


def _sc_diag_body(a_ref, diag_ref, dbuf, diagv, sem):
    wid = lax.axis_index("c") * 16 + lax.axis_index("s")
    r0 = pl.multiple_of(wid * _RPW, _RPW)
    iota16 = lax.iota(jnp.int32, 16)
    zero = jnp.zeros((16,), jnp.float32)

    def per_batch(b, carry):
        pltpu.async_copy(a_ref.at[b, pl.ds(r0, _RPW), pl.ds(r0, _RPW)],
                         dbuf, sem).wait()
        for k in range(_RPW // 16):
            dvec = zero
            for i in range(16):
                v = dbuf[k * 16 + i, pl.ds(k * 16, 16)]
                dvec = dvec + jnp.where(iota16 == i, v, 0.0)
            diagv[pl.ds(k * 16, 16)] = dvec
        pltpu.sync_copy(diagv, diag_ref.at[b, 0, pl.ds(r0, _RPW)])
        return carry

    lax.fori_loop(0, _B, per_batch, 0)


_sc_diag = functools.partial(
    pl.kernel,
    out_type=jax.ShapeDtypeStruct((_B, 8, _N), jnp.float32),
    mesh=plsc.VectorSubcoreMesh(core_axis_name="c", subcore_axis_name="s"),
    scratch_types=[
        pltpu.VMEM((_RPW, _RPW), jnp.float32),
        pltpu.VMEM((_RPW,), jnp.float32),
        pltpu.SemaphoreType.DMA,
    ],
)(_sc_diag_body)


def _phase1_body(a_ref, stats_ref):
    r = pl.program_id(1)
    a = a_ref[0]
    rowsum = jnp.sum(a, axis=1)
    csum = jnp.sum(a, axis=0)

    @pl.when(r == 0)
    def _():
        stats_ref[0, :, 0:1] = jnp.zeros((_N, 1), jnp.float32)

    stats_ref[0, :, 0:1] += csum[:, None]
    stats_ref[0, pl.ds(r * _TR, _TR), 1:2] = rowsum[:, None]


def _phase2_body(x_ref, stats_ref, diag_ref, wn_ref, w2n_ref, wa_ref,
                 sel_ref, binv_ref, psi_ref):
    inv_n = 1.0 / _N
    x = x_ref[0]
    h1 = jnp.dot(x, wn_ref[...], preferred_element_type=jnp.float32)
    sumx = jnp.sum(x, axis=0, keepdims=True)
    base = jnp.dot(sumx, w2n_ref[...], preferred_element_type=jnp.float32)
    cl = stats_ref[0, :, 0:1]
    rw = stats_ref[0, :, 1:2]
    dg = diag_ref[0, 0, :][:, None]
    sum_diag = jnp.sum(dg)
    suma = jnp.sum(rw)
    wa = wa_ref[...]
    base = (base + (sum_diag * inv_n) * wa[1:2]
            + (suma * inv_n * inv_n) * wa[4:5] + wa[5:6])
    pernode = (dg * wa[0:1] + (rw * inv_n) * wa[2:3]
               + (cl * inv_n) * wa[3:4])
    hidden = jnp.maximum(h1 + pernode + base, 0.0)
    s = jnp.sum(hidden, axis=0, keepdims=True)
    psi = jnp.dot(s, sel_ref[...], preferred_element_type=jnp.float32) * inv_n
    psi_ref[0, 0, :] = psi[0] + binv_ref[0]


def kernel(X, A, W_eq, b_eq, W_inv, b_inv):
    n = float(_N)
    Wx = W_eq[:, :, :_F, :]
    wav = W_eq[:, :, _F, :]
    Wn = (Wx[:, 0] + (Wx[:, 2] + Wx[:, 3]) * (1.0 / n))
    Wn = jnp.transpose(Wn, (1, 0, 2)).reshape(_F, _MH)
    W2n = (Wx[:, 1] * (1.0 / n) + Wx[:, 4] * (1.0 / (n * n)))
    W2n = jnp.transpose(W2n, (1, 0, 2)).reshape(_F, _MH)
    wa_rows = [wav[:, p].reshape(_MH) for p in range(5)]
    wa_pack = jnp.stack(wa_rows + [b_eq.reshape(_MH),
                                   jnp.zeros((_MH,), jnp.float32),
                                   jnp.zeros((_MH,), jnp.float32)])
    mh_ids = jnp.arange(_MH, dtype=jnp.int32) // _H
    sel = jnp.where(mh_ids[:, None] == jnp.arange(_M, dtype=jnp.int32)[None, :],
                    W_inv.reshape(_MH)[:, None], 0.0)

    diag = _sc_diag(A)

    stats = pl.pallas_call(
        _phase1_body,
        grid=(_B, _N // _TR),
        in_specs=[pl.BlockSpec((1, _TR, _N), lambda b, r: (b, r, 0))],
        out_specs=pl.BlockSpec((1, _N, 8), lambda b, r: (b, 0, 0)),
        out_shape=jax.ShapeDtypeStruct((_B, _N, 8), jnp.float32),
    )(A)

    psi = pl.pallas_call(
        _phase2_body,
        grid=(_B,),
        in_specs=[
            pl.BlockSpec((1, _N, _F), lambda b: (b, 0, 0)),
            pl.BlockSpec((1, _N, 8), lambda b: (b, 0, 0)),
            pl.BlockSpec((1, 8, _N), lambda b: (b, 0, 0)),
            pl.BlockSpec((_F, _MH), lambda b: (0, 0)),
            pl.BlockSpec((_F, _MH), lambda b: (0, 0)),
            pl.BlockSpec((8, _MH), lambda b: (0, 0)),
            pl.BlockSpec((_MH, _M), lambda b: (0, 0)),
            pl.BlockSpec((1, _M), lambda b: (0, 0)),
        ],
        out_specs=pl.BlockSpec((1, 1, _M), lambda b: (b, 0, 0)),
        out_shape=jax.ShapeDtypeStruct((_B, 1, _M), jnp.float32),
    )(X, stats, diag, Wn, W2n, wa_pack, sel, b_inv.reshape(1, _M))
    return psi.reshape(_B, _M)


# speedup vs baseline: 2.4970x; 1.0237x over previous
import functools

import jax
import jax.numpy as jnp
from jax import lax
from jax.experimental import pallas as pl
from jax.experimental.pallas import tpu as pltpu
from jax.experimental.pallas import tpu_sc as plsc

_B, _N, _F, _M, _H = 4, 4096, 64, 64, 8
_MH = _M * _H
_TR = 1024
_R = _N // _TR
_NW = 32
_RPW = _N // _NW


def _sc_diag_body(a_ref, diag_ref, dbuf, diagv, sem):
    wid = lax.axis_index("c") * 16 + lax.axis_index("s")
    r0 = pl.multiple_of(wid * _RPW, _RPW)
    iota16 = lax.iota(jnp.int32, 16)
    zero = jnp.zeros((16,), jnp.float32)

    def per_batch(b, carry):
        pltpu.async_copy(a_ref.at[b, pl.ds(r0, _RPW), pl.ds(r0, _RPW)],
                         dbuf, sem).wait()
        for k in range(_RPW // 16):
            dvec = zero
            for i in range(16):
                v = dbuf[k * 16 + i, pl.ds(k * 16, 16)]
                dvec = dvec + jnp.where(iota16 == i, v, 0.0)
            diagv[pl.ds(k * 16, 16)] = dvec
        pltpu.sync_copy(diagv, diag_ref.at[b, 0, pl.ds(r0, _RPW)])
        return carry

    lax.fori_loop(0, _B, per_batch, 0)


_sc_diag = functools.partial(
    pl.kernel,
    out_type=jax.ShapeDtypeStruct((_B, 8, _N), jnp.float32),
    mesh=plsc.VectorSubcoreMesh(core_axis_name="c", subcore_axis_name="s"),
    scratch_types=[
        pltpu.VMEM((_RPW, _RPW), jnp.float32),
        pltpu.VMEM((_RPW,), jnp.float32),
        pltpu.SemaphoreType.DMA,
    ],
)(_sc_diag_body)


def _fused_body(a_ref, x_ref, diag_ref, wn_ref, w2n_ref, wa_ref, sel_ref,
                binv_ref, psi_ref, stats_ref):
    r = pl.program_id(1)

    @pl.when(r < _R)
    def _():
        a = a_ref[0]
        rowsum = jnp.sum(a, axis=1)
        csum = jnp.sum(a, axis=0)

        @pl.when(r == 0)
        def _():
            stats_ref[:, 0:1] = jnp.zeros((_N, 1), jnp.float32)

        stats_ref[:, 0:1] += csum[:, None]
        stats_ref[pl.ds(r * _TR, _TR), 1:2] = rowsum[:, None]

    @pl.when(r == _R)
    def _():
        inv_n = 1.0 / _N
        x = x_ref[0]
        h1 = jnp.dot(x, wn_ref[...], preferred_element_type=jnp.float32)
        sumx = jnp.sum(x, axis=0, keepdims=True)
        base = jnp.dot(sumx, w2n_ref[...], preferred_element_type=jnp.float32)
        cl = stats_ref[:, 0:1]
        rw = stats_ref[:, 1:2]
        dg = diag_ref[0, 0, :][:, None]
        sum_diag = jnp.sum(dg)
        suma = jnp.sum(rw)
        wa = wa_ref[...]
        base = (base + (sum_diag * inv_n) * wa[1:2]
                + (suma * inv_n * inv_n) * wa[4:5] + wa[5:6])
        pernode = (dg * wa[0:1] + (rw * inv_n) * wa[2:3]
                   + (cl * inv_n) * wa[3:4])
        hidden = jnp.maximum(h1 + pernode + base, 0.0)
        s = jnp.sum(hidden, axis=0, keepdims=True)
        psi = jnp.dot(s, sel_ref[...],
                      preferred_element_type=jnp.float32) * inv_n
        psi_ref[0, 0, :] = psi[0] + binv_ref[0]


def kernel(X, A, W_eq, b_eq, W_inv, b_inv):
    n = float(_N)
    Wx = W_eq[:, :, :_F, :]
    wav = W_eq[:, :, _F, :]
    Wn = (Wx[:, 0] + (Wx[:, 2] + Wx[:, 3]) * (1.0 / n))
    Wn = jnp.transpose(Wn, (1, 0, 2)).reshape(_F, _MH)
    W2n = (Wx[:, 1] * (1.0 / n) + Wx[:, 4] * (1.0 / (n * n)))
    W2n = jnp.transpose(W2n, (1, 0, 2)).reshape(_F, _MH)
    wa_rows = [wav[:, p].reshape(_MH) for p in range(5)]
    wa_pack = jnp.stack(wa_rows + [b_eq.reshape(_MH),
                                   jnp.zeros((_MH,), jnp.float32),
                                   jnp.zeros((_MH,), jnp.float32)])
    mh_ids = jnp.arange(_MH, dtype=jnp.int32) // _H
    sel = jnp.where(mh_ids[:, None] == jnp.arange(_M, dtype=jnp.int32)[None, :],
                    W_inv.reshape(_MH)[:, None], 0.0)

    diag = _sc_diag(A)

    psi = pl.pallas_call(
        _fused_body,
        grid=(_B, _R + 1),
        in_specs=[
            pl.BlockSpec((1, _TR, _N),
                         lambda b, r: (b, jnp.minimum(r, _R - 1), 0)),
            pl.BlockSpec((1, _N, _F), lambda b, r: (b, 0, 0)),
            pl.BlockSpec((1, 8, _N), lambda b, r: (b, 0, 0)),
            pl.BlockSpec((_F, _MH), lambda b, r: (0, 0)),
            pl.BlockSpec((_F, _MH), lambda b, r: (0, 0)),
            pl.BlockSpec((8, _MH), lambda b, r: (0, 0)),
            pl.BlockSpec((_MH, _M), lambda b, r: (0, 0)),
            pl.BlockSpec((1, _M), lambda b, r: (0, 0)),
        ],
        out_specs=pl.BlockSpec((1, 1, _M), lambda b, r: (b, 0, 0)),
        out_shape=jax.ShapeDtypeStruct((_B, 1, _M), jnp.float32),
        scratch_shapes=[pltpu.VMEM((_N, 8), jnp.float32)],
    )(A, X, diag, Wn, W2n, wa_pack, sel, b_inv.reshape(1, _M))
    return psi.reshape(_B, _M)
